# baseline (speedup 1.0000x reference)
import functools

import jax
import jax.numpy as jnp
from jax import lax
from jax.experimental import pallas as pl
from jax.experimental.pallas import tpu as pltpu
from jax.experimental.pallas import tpu_sc as plsc

EMBED = 128
LANES = 16
CHUNK = 128


def _lane_sum(x):
    idx = lax.iota(jnp.int32, LANES)
    dnums = lax.GatherDimensionNumbers(
        offset_dims=(), collapsed_slice_dims=(0,), start_index_map=(0,))
    for k in (8, 4, 2, 1):
        perm = jnp.bitwise_xor(idx, k).reshape(LANES, 1)
        x = x + lax.gather(x, perm, dnums, slice_sizes=(1,),
                           mode=lax.GatherScatterMode.PROMISE_IN_BOUNDS)
    return x


def _rsqrt_vec(v):
    i = lax.bitcast_convert_type(v, jnp.int32)
    i = jnp.int32(0x5F3759DF) - lax.shift_right_logical(i, 1)
    y = lax.bitcast_convert_type(i, jnp.float32)
    for _ in range(3):
        y = y * (1.5 - 0.5 * v * y * y)
    return y


@functools.lru_cache(maxsize=None)
def _build(n_tok, seq_len):
    info = plsc.get_sparse_core_info()
    nc, ns = info.num_cores, info.num_subcores
    nw = nc * ns
    tok_w = n_tok // nw
    n_chunks = tok_w // CHUNK
    assert tok_w * nw == n_tok and n_chunks * CHUNK == tok_w
    assert seq_len % tok_w == 0
    nj = EMBED // LANES

    mesh = plsc.VectorSubcoreMesh(core_axis_name="c", subcore_axis_name="s")

    @functools.partial(
        pl.kernel,
        mesh=mesh,
        out_type=jax.ShapeDtypeStruct((n_tok, EMBED), jnp.float32),
        scratch_types=[
            pltpu.VMEM((n_chunks, CHUNK), jnp.int32),
            pltpu.VMEM((n_chunks, CHUNK), jnp.int32),
            pltpu.VMEM((tok_w, EMBED), jnp.float32),
            pltpu.VMEM((tok_w, EMBED), jnp.float32),
            pltpu.VMEM((tok_w, EMBED), jnp.float32),
            pltpu.VMEM((EMBED,), jnp.float32),
            pltpu.VMEM((EMBED,), jnp.float32),
            pltpu.SemaphoreType.DMA,
        ],
    )
    def emb_kernel(ids_hbm, tt_hbm, word_hbm, type_hbm, pos_hbm, g_hbm,
                   b_hbm, out_hbm, idx_v, tt_v, w_v, t_v, p_v, g_v, b_v,
                   sem):
        wid = lax.axis_index("s") * nc + lax.axis_index("c")
        base = wid * tok_w
        pltpu.sync_copy(ids_hbm.at[pl.ds(wid * n_chunks, n_chunks)], idx_v)
        pltpu.sync_copy(tt_hbm.at[pl.ds(wid * n_chunks, n_chunks)], tt_v)
        copies = []
        for c in range(n_chunks):
            copies.append(pltpu.async_copy(
                word_hbm.at[idx_v.at[c]],
                w_v.at[pl.ds(c * CHUNK, CHUNK)], sem))
            copies.append(pltpu.async_copy(
                type_hbm.at[tt_v.at[c]],
                t_v.at[pl.ds(c * CHUNK, CHUNK)], sem))
        s_base = lax.rem(base, seq_len)
        pltpu.sync_copy(pos_hbm.at[pl.ds(s_base, tok_w)], p_v)
        pltpu.sync_copy(g_hbm, g_v)
        pltpu.sync_copy(b_hbm, b_v)
        for cp in copies:
            cp.wait()

        gs = [g_v[pl.ds(j * LANES, LANES)] for j in range(nj)]
        bs = [b_v[pl.ds(j * LANES, LANES)] for j in range(nj)]
        inv_n = 1.0 / EMBED

        def body(t, carry):
            xs = []
            s = jnp.zeros((LANES,), jnp.float32)
            for j in range(nj):
                sl = pl.ds(j * LANES, LANES)
                x = w_v[t, sl] + t_v[t, sl] + p_v[t, sl]
                xs.append(x)
                s = s + x
            mean = _lane_sum(s) * inv_n
            ss = jnp.zeros((LANES,), jnp.float32)
            dv = []
            for j in range(nj):
                d = xs[j] - mean
                dv.append(d)
                ss = ss + d * d
            var = _lane_sum(ss) * inv_n
            r = _rsqrt_vec(var + 1e-12)
            for j in range(nj):
                w_v[t, pl.ds(j * LANES, LANES)] = dv[j] * r * gs[j] + bs[j]
            return carry

        lax.fori_loop(0, tok_w, body, jnp.int32(0))
        pltpu.sync_copy(w_v, out_hbm.at[pl.ds(base, tok_w)])

    return emb_kernel


def kernel(input_ids, token_type_ids, word_table, type_table, pos_table,
           ln_gamma, ln_beta):
    b, s = input_ids.shape
    n_tok = b * s
    ids = input_ids.reshape(-1, CHUNK).astype(jnp.int32)
    tts = token_type_ids.reshape(-1, CHUNK).astype(jnp.int32)
    fn = _build(n_tok, s)
    out = fn(ids, tts, word_table.astype(jnp.float32),
             type_table.astype(jnp.float32), pos_table.astype(jnp.float32),
             ln_gamma.astype(jnp.float32), ln_beta.astype(jnp.float32))
    return out.reshape(b, s, EMBED)


# speedup vs baseline: 2.0335x; 2.0335x over previous
import functools

import jax
import jax.numpy as jnp
from jax import lax
from jax.experimental import pallas as pl
from jax.experimental.pallas import tpu as pltpu
from jax.experimental.pallas import tpu_sc as plsc

EMBED = 128
LANES = 16
CHUNK = 128
UNROLL = 8


def _rsqrt_vec(v):
    i = lax.bitcast_convert_type(v, jnp.int32)
    i = jnp.int32(0x5F3759DF) - lax.shift_right_logical(i, 1)
    y = lax.bitcast_convert_type(i, jnp.float32)
    for _ in range(3):
        y = y * (1.5 - 0.5 * v * y * y)
    return y


@functools.lru_cache(maxsize=None)
def _build(n_tok, seq_len):
    info = plsc.get_sparse_core_info()
    nc, ns = info.num_cores, info.num_subcores
    nw = nc * ns
    tok_w = n_tok // nw
    n_chunks = tok_w // CHUNK
    n_groups = tok_w // LANES
    assert tok_w * nw == n_tok and n_chunks * CHUNK == tok_w
    assert seq_len % tok_w == 0

    mesh = plsc.VectorSubcoreMesh(core_axis_name="c", subcore_axis_name="s")

    @functools.partial(
        pl.kernel,
        mesh=mesh,
        compiler_params=pltpu.CompilerParams(needs_layout_passes=False),
        out_type=jax.ShapeDtypeStruct((n_tok * EMBED,), jnp.float32),
        scratch_types=[
            pltpu.VMEM((n_chunks, CHUNK), jnp.int32),
            pltpu.VMEM((tok_w,), jnp.int32),
            pltpu.VMEM((tok_w, EMBED), jnp.float32),
            pltpu.VMEM((tok_w * EMBED,), jnp.float32),
            pltpu.VMEM((tok_w * EMBED,), jnp.float32),
            pltpu.VMEM((2 * EMBED,), jnp.float32),
            pltpu.VMEM((EMBED * LANES,), jnp.float32),
            pltpu.VMEM((EMBED,), jnp.float32),
            pltpu.VMEM((EMBED,), jnp.float32),
            pltpu.SemaphoreType.DMA,
        ],
    )
    def emb_kernel(ids_hbm, tt_hbm, word_hbm, type_hbm, pos_hbm, g_hbm,
                   b_hbm, out_hbm, idx_v, tt_v, w_v, p_v, o_v, ttab_v, xT,
                   g_v, b_v, sem):
        wid = lax.axis_index("s") * nc + lax.axis_index("c")
        base = wid * tok_w
        pltpu.sync_copy(ids_hbm.at[pl.ds(wid * n_chunks, n_chunks)], idx_v)
        copies = []
        for c in range(n_chunks):
            copies.append(pltpu.async_copy(
                word_hbm.at[idx_v.at[c]],
                w_v.at[pl.ds(c * CHUNK, CHUNK)], sem))
        s_base = lax.rem(base, seq_len)
        pltpu.sync_copy(pos_hbm.at[pl.ds(s_base * EMBED, tok_w * EMBED)],
                        p_v)
        pltpu.sync_copy(tt_hbm.at[pl.ds(base, tok_w)], tt_v)
        pltpu.sync_copy(type_hbm, ttab_v)
        pltpu.sync_copy(g_hbm, g_v)
        pltpu.sync_copy(b_hbm, b_v)
        for cp in copies:
            cp.wait()

        lane = lax.iota(jnp.int32, LANES)
        lane16 = lane * LANES
        inv_n = 1.0 / EMBED
        zero = jnp.zeros((LANES,), jnp.float32)

        def group(g, carry):
            t0 = g * LANES
            tt16 = tt_v[pl.ds(t0, LANES)] * EMBED
            row_idx = (lax.broadcast(t0, (LANES,)) + lane) * EMBED

            def transpose(t, c):
                tok = t0 + t
                for j in range(EMBED // LANES):
                    sl = pl.ds(j * LANES, LANES)
                    v = w_v[tok, sl] + p_v[pl.ds(tok * EMBED + j * LANES,
                                                 LANES)]
                    plsc.store_scatter(xT, [lane16 + (j * 256 + t)], v)
                return c

            lax.fori_loop(0, LANES, transpose, jnp.int32(0))

            def stats(eb, c):
                s, ss = c
                for k in range(UNROLL):
                    e = eb * UNROLL + k
                    sl = pl.ds(e * LANES, LANES)
                    x = xT[sl] + plsc.load_gather(
                        ttab_v, [tt16 + lax.broadcast(e, (LANES,))])
                    xT[sl] = x
                    s = s + x
                    ss = ss + x * x
                return s, ss

            s, ss = lax.fori_loop(0, EMBED // UNROLL, stats, (zero, zero))
            mean = s * inv_n
            var = jnp.maximum(ss * inv_n - mean * mean, 0.0)
            r = _rsqrt_vec(var + 1e-12)
            mr = mean * r

            def norm(eb, c):
                for k in range(UNROLL):
                    e = eb * UNROLL + k
                    ev = lax.broadcast(e, (LANES,))
                    ge = plsc.load_gather(g_v, [ev])
                    be = plsc.load_gather(b_v, [ev])
                    o = (xT[pl.ds(e * LANES, LANES)] * r - mr) * ge + be
                    plsc.store_scatter(o_v, [row_idx + ev], o)
                return c

            lax.fori_loop(0, EMBED // UNROLL, norm, jnp.int32(0))
            return carry

        lax.fori_loop(0, n_groups, group, jnp.int32(0))
        pltpu.sync_copy(o_v, out_hbm.at[pl.ds(base * EMBED, tok_w * EMBED)])

    return emb_kernel


def kernel(input_ids, token_type_ids, word_table, type_table, pos_table,
           ln_gamma, ln_beta):
    b, s = input_ids.shape
    n_tok = b * s
    ids = input_ids.reshape(-1, CHUNK).astype(jnp.int32)
    tts = token_type_ids.reshape(-1).astype(jnp.int32)
    fn = _build(n_tok, s)
    out = fn(ids, tts, word_table.astype(jnp.float32),
             type_table.astype(jnp.float32).reshape(-1),
             pos_table.astype(jnp.float32).reshape(-1),
             ln_gamma.astype(jnp.float32), ln_beta.astype(jnp.float32))
    return out.reshape(b, s, EMBED)


# speedup vs baseline: 2.1586x; 1.0615x over previous
import functools

import jax
import jax.numpy as jnp
from jax import lax
from jax.experimental import pallas as pl
from jax.experimental.pallas import tpu as pltpu
from jax.experimental.pallas import tpu_sc as plsc

EMBED = 128
LANES = 16
CHUNK = 128
UNROLL = 8
XSTRIDE = LANES + 1


def _rsqrt_vec(v):
    i = lax.bitcast_convert_type(v, jnp.int32)
    i = jnp.int32(0x5F3759DF) - lax.shift_right_logical(i, 1)
    y = lax.bitcast_convert_type(i, jnp.float32)
    for _ in range(3):
        y = y * (1.5 - 0.5 * v * y * y)
    return y


@functools.lru_cache(maxsize=None)
def _build(n_tok, seq_len):
    info = plsc.get_sparse_core_info()
    nc, ns = info.num_cores, info.num_subcores
    nw = nc * ns
    tok_w = n_tok // nw
    n_chunks = tok_w // CHUNK
    n_groups = tok_w // LANES
    nj = EMBED // LANES
    assert tok_w * nw == n_tok and n_chunks * CHUNK == tok_w
    assert seq_len % tok_w == 0

    mesh = plsc.VectorSubcoreMesh(core_axis_name="c", subcore_axis_name="s")

    @functools.partial(
        pl.kernel,
        mesh=mesh,
        compiler_params=pltpu.CompilerParams(needs_layout_passes=False),
        out_type=jax.ShapeDtypeStruct((n_tok, EMBED), jnp.float32),
        scratch_types=[
            pltpu.VMEM((n_chunks, CHUNK), jnp.int32),
            pltpu.VMEM((tok_w,), jnp.int32),
            pltpu.VMEM((tok_w, EMBED), jnp.float32),
            pltpu.VMEM((tok_w * EMBED,), jnp.float32),
            pltpu.VMEM((2 * EMBED,), jnp.float32),
            pltpu.VMEM((EMBED * XSTRIDE,), jnp.float32),
            pltpu.VMEM((EMBED,), jnp.float32),
            pltpu.VMEM((EMBED,), jnp.float32),
            pltpu.SemaphoreType.DMA,
        ],
    )
    def emb_kernel(ids_hbm, tt_hbm, word_hbm, type_hbm, pos_hbm, g_hbm,
                   b_hbm, out_hbm, idx_v, tt_v, w_v, p_v, ttab_v, xT, g_v,
                   b_v, sem):
        wid = lax.axis_index("s") * nc + lax.axis_index("c")
        base = wid * tok_w
        pltpu.sync_copy(ids_hbm.at[pl.ds(wid * n_chunks, n_chunks)], idx_v)
        copies = []
        for c in range(n_chunks):
            copies.append(pltpu.async_copy(
                word_hbm.at[idx_v.at[c]],
                w_v.at[pl.ds(c * CHUNK, CHUNK)], sem))
        s_base = lax.rem(base, seq_len)
        pltpu.sync_copy(pos_hbm.at[pl.ds(s_base * EMBED, tok_w * EMBED)],
                        p_v)
        pltpu.sync_copy(tt_hbm.at[pl.ds(base, tok_w)], tt_v)
        pltpu.sync_copy(type_hbm, ttab_v)
        pltpu.sync_copy(g_hbm, g_v)
        pltpu.sync_copy(b_hbm, b_v)
        for cp in copies:
            cp.wait()

        lane = lax.iota(jnp.int32, LANES)
        lane17 = lane * XSTRIDE
        inv_n = 1.0 / EMBED
        zero = jnp.zeros((LANES,), jnp.float32)
        gj = [g_v[pl.ds(j * LANES, LANES)] for j in range(nj)]
        bj = [b_v[pl.ds(j * LANES, LANES)] for j in range(nj)]

        def group(g, carry):
            t0 = g * LANES
            tt16 = tt_v[pl.ds(t0, LANES)] * EMBED

            def transpose_in(t, c):
                tok = t0 + t
                for j in range(nj):
                    v = (w_v[tok, pl.ds(j * LANES, LANES)]
                         + p_v[pl.ds(tok * EMBED + j * LANES, LANES)])
                    plsc.store_scatter(
                        xT, [lane17 + (j * (LANES * XSTRIDE) + t)], v)
                return c

            lax.fori_loop(0, LANES, transpose_in, jnp.int32(0))

            def stats(eb, c):
                s, ss = c
                for k in range(UNROLL):
                    e = eb * UNROLL + k
                    exi = lane + (e * XSTRIDE)
                    x = (plsc.load_gather(xT, [exi])
                         + plsc.load_gather(
                             ttab_v, [tt16 + lax.broadcast(e, (LANES,))]))
                    plsc.store_scatter(xT, [exi], x)
                    s = s + x
                    ss = ss + x * x
                return s, ss

            s, ss = lax.fori_loop(0, EMBED // UNROLL, stats, (zero, zero))
            mean = s * inv_n
            var = jnp.maximum(ss * inv_n - mean * mean, 0.0)
            r = _rsqrt_vec(var + 1e-12)
            mr = mean * r

            def norm(eb, c):
                for k in range(UNROLL):
                    e = eb * UNROLL + k
                    exi = lane + (e * XSTRIDE)
                    plsc.store_scatter(
                        xT, [exi], plsc.load_gather(xT, [exi]) * r - mr)
                return c

            lax.fori_loop(0, EMBED // UNROLL, norm, jnp.int32(0))

            def transpose_out(t, c):
                tok = t0 + t
                for j in range(nj):
                    v = plsc.load_gather(
                        xT, [lane17 + (j * (LANES * XSTRIDE) + t)])
                    w_v[tok, pl.ds(j * LANES, LANES)] = v * gj[j] + bj[j]
                return c

            lax.fori_loop(0, LANES, transpose_out, jnp.int32(0))
            return carry

        lax.fori_loop(0, n_groups, group, jnp.int32(0))
        pltpu.sync_copy(w_v, out_hbm.at[pl.ds(base, tok_w)])

    return emb_kernel


def kernel(input_ids, token_type_ids, word_table, type_table, pos_table,
           ln_gamma, ln_beta):
    b, s = input_ids.shape
    n_tok = b * s
    ids = input_ids.reshape(-1, CHUNK).astype(jnp.int32)
    tts = token_type_ids.reshape(-1).astype(jnp.int32)
    fn = _build(n_tok, s)
    out = fn(ids, tts, word_table.astype(jnp.float32),
             type_table.astype(jnp.float32).reshape(-1),
             pos_table.astype(jnp.float32).reshape(-1),
             ln_gamma.astype(jnp.float32), ln_beta.astype(jnp.float32))
    return out.reshape(b, s, EMBED)


# speedup vs baseline: 2.7051x; 1.2532x over previous
import functools

import jax
import jax.numpy as jnp
from jax import lax
from jax.experimental import pallas as pl
from jax.experimental.pallas import tpu as pltpu
from jax.experimental.pallas import tpu_sc as plsc

EMBED = 128
LANES = 16
CHUNK = 128
XSTRIDE = LANES + 1


def _rsqrt_vec(v):
    i = lax.bitcast_convert_type(v, jnp.int32)
    i = jnp.int32(0x5F3759DF) - lax.shift_right_logical(i, 1)
    y = lax.bitcast_convert_type(i, jnp.float32)
    for _ in range(3):
        y = y * (1.5 - 0.5 * v * y * y)
    return y


@functools.lru_cache(maxsize=None)
def _build(n_tok, seq_len):
    info = plsc.get_sparse_core_info()
    nc, ns = info.num_cores, info.num_subcores
    nw = nc * ns
    tok_w = n_tok // nw
    n_chunks = tok_w // CHUNK
    n_groups = tok_w // LANES
    nj = EMBED // LANES
    assert tok_w * nw == n_tok and n_chunks * CHUNK == tok_w
    assert seq_len % tok_w == 0

    mesh = plsc.VectorSubcoreMesh(core_axis_name="c", subcore_axis_name="s")

    @functools.partial(
        pl.kernel,
        mesh=mesh,
        compiler_params=pltpu.CompilerParams(needs_layout_passes=False),
        out_type=jax.ShapeDtypeStruct((n_tok, EMBED), jnp.float32),
        scratch_types=[
            pltpu.VMEM((n_chunks, CHUNK), jnp.int32),
            pltpu.VMEM((tok_w,), jnp.int32),
            pltpu.VMEM((tok_w, EMBED), jnp.float32),
            pltpu.VMEM((tok_w * EMBED,), jnp.float32),
            pltpu.VMEM((2 * EMBED,), jnp.float32),
            pltpu.VMEM((EMBED * XSTRIDE,), jnp.float32),
            pltpu.VMEM((LANES,), jnp.float32),
            pltpu.VMEM((LANES,), jnp.float32),
            pltpu.VMEM((EMBED,), jnp.float32),
            pltpu.VMEM((EMBED,), jnp.float32),
            pltpu.SemaphoreType.DMA,
        ],
    )
    def emb_kernel(ids_hbm, tt_hbm, word_hbm, type_hbm, pos_hbm, g_hbm,
                   b_hbm, out_hbm, idx_v, tt_v, w_v, p_v, ttab_v, xT, r_v,
                   m_v, g_v, b_v, sem):
        wid = lax.axis_index("s") * nc + lax.axis_index("c")
        base = wid * tok_w
        pltpu.sync_copy(ids_hbm.at[pl.ds(wid * n_chunks, n_chunks)], idx_v)
        copies = []
        for c in range(n_chunks):
            copies.append(pltpu.async_copy(
                word_hbm.at[idx_v.at[c]],
                w_v.at[pl.ds(c * CHUNK, CHUNK)], sem))
        s_base = lax.rem(base, seq_len)
        pltpu.sync_copy(pos_hbm.at[pl.ds(s_base * EMBED, tok_w * EMBED)],
                        p_v)
        pltpu.sync_copy(tt_hbm.at[pl.ds(base, tok_w)], tt_v)
        pltpu.sync_copy(type_hbm, ttab_v)
        pltpu.sync_copy(g_hbm, g_v)
        pltpu.sync_copy(b_hbm, b_v)
        for cp in copies:
            cp.wait()

        lane = lax.iota(jnp.int32, LANES)
        lane17 = lane * XSTRIDE
        inv_n = 1.0 / EMBED
        zero = jnp.zeros((LANES,), jnp.float32)
        gj = [g_v[pl.ds(j * LANES, LANES)] for j in range(nj)]
        bj = [b_v[pl.ds(j * LANES, LANES)] for j in range(nj)]

        def group(g, carry):
            t0 = g * LANES
            tok16 = lax.broadcast(t0, (LANES,)) + lane
            row_idx = tok16 * EMBED
            tt16 = tt_v[pl.ds(t0, LANES)] * EMBED

            @plsc.parallel_loop(0, EMBED, unroll=8, carry=(zero, zero))
            def stats(e, c):
                s, ss = c
                ev = lax.broadcast(e, (LANES,))
                x = (plsc.load_gather(w_v, [tok16, ev])
                     + plsc.load_gather(p_v, [row_idx + ev])
                     + plsc.load_gather(ttab_v, [tt16 + ev]))
                plsc.store_scatter(xT, [lane + e * XSTRIDE], x)
                return s + x, ss + x * x

            s, ss = stats
            mean = s * inv_n
            var = jnp.maximum(ss * inv_n - mean * mean, 0.0)
            r = _rsqrt_vec(var + 1e-12)
            r_v[...] = r
            m_v[...] = mean * r

            @plsc.parallel_loop(0, LANES, unroll=2)
            def transpose_out(t):
                tv = lax.broadcast(t, (LANES,))
                rb = plsc.load_gather(r_v, [tv])
                mb = plsc.load_gather(m_v, [tv])
                tok = t0 + t
                for j in range(nj):
                    v = plsc.load_gather(
                        xT, [lane17 + (j * (LANES * XSTRIDE) + t)])
                    w_v[tok, pl.ds(j * LANES, LANES)] = (
                        (v * rb - mb) * gj[j] + bj[j])

            return carry

        lax.fori_loop(0, n_groups, group, jnp.int32(0))
        pltpu.sync_copy(w_v, out_hbm.at[pl.ds(base, tok_w)])

    return emb_kernel


def kernel(input_ids, token_type_ids, word_table, type_table, pos_table,
           ln_gamma, ln_beta):
    b, s = input_ids.shape
    n_tok = b * s
    ids = input_ids.reshape(-1, CHUNK).astype(jnp.int32)
    tts = token_type_ids.reshape(-1).astype(jnp.int32)
    fn = _build(n_tok, s)
    out = fn(ids, tts, word_table.astype(jnp.float32),
             type_table.astype(jnp.float32).reshape(-1),
             pos_table.astype(jnp.float32).reshape(-1),
             ln_gamma.astype(jnp.float32), ln_beta.astype(jnp.float32))
    return out.reshape(b, s, EMBED)


# speedup vs baseline: 5.6680x; 2.0953x over previous
import functools

import jax
import jax.numpy as jnp
from jax import lax
from jax.experimental import pallas as pl
from jax.experimental.pallas import tpu as pltpu
from jax.experimental.pallas import tpu_sc as plsc

EMBED = 128
LANES = 16
CHUNK = 128

_DNUMS = lax.GatherDimensionNumbers(
    offset_dims=(), collapsed_slice_dims=(0,), start_index_map=(0,))


def _shuffle(x, idx):
    return lax.gather(x, idx.reshape(LANES, 1), _DNUMS, slice_sizes=(1,),
                      mode=lax.GatherScatterMode.PROMISE_IN_BOUNDS)


def _lane_sum(x, lane):
    for k in (8, 4, 2, 1):
        x = x + _shuffle(x, jnp.bitwise_xor(lane, k))
    return x


def _rsqrt_vec(v):
    i = lax.bitcast_convert_type(v, jnp.int32)
    i = jnp.int32(0x5F3759DF) - lax.shift_right_logical(i, 1)
    y = lax.bitcast_convert_type(i, jnp.float32)
    for _ in range(3):
        y = y * (1.5 - 0.5 * v * y * y)
    return y


@functools.lru_cache(maxsize=None)
def _build(n_tok, seq_len):
    info = plsc.get_sparse_core_info()
    nc, ns = info.num_cores, info.num_subcores
    nw = nc * ns
    tok_w = n_tok // nw
    n_chunks = tok_w // CHUNK
    n_groups = tok_w // LANES
    nj = EMBED // LANES
    assert tok_w * nw == n_tok and n_chunks * CHUNK == tok_w
    assert seq_len % tok_w == 0

    mesh = plsc.VectorSubcoreMesh(core_axis_name="c", subcore_axis_name="s")

    @functools.partial(
        pl.kernel,
        mesh=mesh,
        compiler_params=pltpu.CompilerParams(needs_layout_passes=False),
        out_type=jax.ShapeDtypeStruct((n_tok, EMBED), jnp.float32),
        scratch_types=[
            pltpu.VMEM((n_chunks, CHUNK), jnp.int32),
            pltpu.VMEM((tok_w,), jnp.int32),
            pltpu.VMEM((tok_w, EMBED), jnp.float32),
            pltpu.VMEM((tok_w * EMBED,), jnp.float32),
            pltpu.VMEM((2 * EMBED,), jnp.float32),
            pltpu.VMEM((EMBED,), jnp.float32),
            pltpu.VMEM((EMBED,), jnp.float32),
            pltpu.SemaphoreType.DMA,
            pltpu.SemaphoreType.DMA,
        ],
    )
    def emb_kernel(ids_hbm, tt_hbm, word_hbm, type_hbm, pos_hbm, g_hbm,
                   b_hbm, out_hbm, idx_v, tt_v, w_v, p_v, ttab_v, g_v, b_v,
                   sem, sem2):
        wid = lax.axis_index("s") * nc + lax.axis_index("c")
        base = wid * tok_w
        pltpu.sync_copy(ids_hbm.at[pl.ds(wid * n_chunks, n_chunks)], idx_v)
        copies = []
        for c in range(n_chunks):
            copies.append(pltpu.async_copy(
                word_hbm.at[idx_v.at[c]],
                w_v.at[pl.ds(c * CHUNK, CHUNK)], sem))
        s_base = lax.rem(base, seq_len)
        copies.append(pltpu.async_copy(
            pos_hbm.at[pl.ds(s_base * EMBED, tok_w * EMBED)], p_v, sem2))
        copies.append(pltpu.async_copy(
            tt_hbm.at[pl.ds(base, tok_w)], tt_v, sem2))
        copies.append(pltpu.async_copy(type_hbm, ttab_v, sem2))
        copies.append(pltpu.async_copy(g_hbm, g_v, sem2))
        copies.append(pltpu.async_copy(b_hbm, b_v, sem2))
        for cp in copies:
            cp.wait()

        lane = lax.iota(jnp.int32, LANES)
        inv_n = 1.0 / EMBED
        t0j = [ttab_v[pl.ds(j * LANES, LANES)] for j in range(nj)]
        d1j = [ttab_v[pl.ds(EMBED + j * LANES, LANES)] - t0j[j]
               for j in range(nj)]
        gj = [g_v[pl.ds(j * LANES, LANES)] for j in range(nj)]
        bj = [b_v[pl.ds(j * LANES, LANES)] for j in range(nj)]

        def group(g, carry):
            t0 = g * LANES
            tt16f = tt_v[pl.ds(t0, LANES)].astype(jnp.float32)

            @plsc.parallel_loop(0, LANES, unroll=2)
            def token(t):
                tok = t0 + t
                f = _shuffle(tt16f, lax.broadcast(t, (LANES,)))
                xs = []
                for j in range(nj):
                    x = (w_v[tok, pl.ds(j * LANES, LANES)]
                         + p_v[pl.ds(tok * EMBED + j * LANES, LANES)]
                         + t0j[j] + f * d1j[j])
                    xs.append(x)
                u = xs[0]
                u2 = xs[0] * xs[0]
                for j in range(1, nj):
                    u = u + xs[j]
                    u2 = u2 + xs[j] * xs[j]
                ssum = _lane_sum(u, lane)
                ssq = _lane_sum(u2, lane)
                mean = ssum * inv_n
                var = jnp.maximum(ssq * inv_n - mean * mean, 0.0)
                r = _rsqrt_vec(var + 1e-12)
                mr = mean * r
                for j in range(nj):
                    w_v[tok, pl.ds(j * LANES, LANES)] = (
                        (xs[j] * r - mr) * gj[j] + bj[j])

            return carry

        lax.fori_loop(0, n_groups, group, jnp.int32(0))
        pltpu.sync_copy(w_v, out_hbm.at[pl.ds(base, tok_w)])

    return emb_kernel


def kernel(input_ids, token_type_ids, word_table, type_table, pos_table,
           ln_gamma, ln_beta):
    b, s = input_ids.shape
    n_tok = b * s
    ids = input_ids.reshape(-1, CHUNK).astype(jnp.int32)
    tts = token_type_ids.reshape(-1).astype(jnp.int32)
    fn = _build(n_tok, s)
    out = fn(ids, tts, word_table.astype(jnp.float32),
             type_table.astype(jnp.float32).reshape(-1),
             pos_table.astype(jnp.float32).reshape(-1),
             ln_gamma.astype(jnp.float32), ln_beta.astype(jnp.float32))
    return out.reshape(b, s, EMBED)
